# Initial kernel scaffold; baseline (speedup 1.0000x reference)
import jax
import jax.numpy as jnp
from jax.experimental import pallas as pl


def kernel(node_repr, rel_repr, edge_norm, hyper_plane_node_w, hyper_plane_node_rel_w, hyper_plane_in_w, hyper_plane_out_w, loop_w, loop_rel, att_w, w_rel, bias, bn_gamma, bn_beta, edge_index, edge_type):
    raise NotImplementedError("write your pallas kernel here")


# speedup vs baseline: 3.6229x; 3.6229x over previous
import functools

import jax
import jax.numpy as jnp
from jax import lax
from jax.experimental import pallas as pl
from jax.experimental.pallas import tpu as pltpu
from jax.experimental.pallas import tpu_sc as plsc

F32 = jnp.float32

V = 10000
E = 320000
IN_C = 128
OUT_C = 128
K = 4
NUM_REL2 = 64
D = IN_C // K

NC = 2
NS = 16
NW = NC * NS
E_PER_W = E // NW
CHUNK = 80
N_CHUNK = E_PER_W // CHUNK
V_PAD = 10240
V_PER_T = V_PAD // NS
ZROWS = 128

EDGE_B = 1600
N_EB = E // EDGE_B


def _prep_body(node_ref, wn_ref, a2_ref, rel_ref, wrel_ref, s16_ref, relo_ref):
    p = jnp.dot(node_ref[...], wn_ref[...], preferred_element_type=F32)
    s16_ref[...] = jnp.dot(p, a2_ref[...], preferred_element_type=F32)
    relo_ref[...] = jnp.dot(rel_ref[...], wrel_ref[...], preferred_element_type=F32)


def _gather_body(node_hbm, s16_hbm, src_hbm, dst_hbm, hsrc_hbm, sa_hbm,
                 sidx_v, didx_v, rows_v, srows_v, sem):
    wid = lax.axis_index("c") * NS + lax.axis_index("s")
    ebase = wid * E_PER_W

    def step(j, carry):
        base = ebase + j * CHUNK
        pltpu.sync_copy(src_hbm.at[pl.ds(base, CHUNK)], sidx_v)
        pltpu.async_copy(node_hbm.at[sidx_v], rows_v, sem).wait()
        pltpu.sync_copy(rows_v, hsrc_hbm.at[pl.ds(base, CHUNK)])
        pltpu.sync_copy(dst_hbm.at[pl.ds(base, CHUNK)], didx_v)
        pltpu.async_copy(s16_hbm.at[didx_v], srows_v, sem).wait()
        pltpu.sync_copy(srows_v, sa_hbm.at[pl.ds(base, CHUNK)])
        return carry

    lax.fori_loop(0, N_CHUNK, step, 0)


def _edge_body(hsrc_ref, sa_ref, aux_ref, rel_ref, wnr_ref, a1_ref, rrep_ref,
               bd_ref, out_ref):
    hs = hsrc_ref[...]
    et = aux_ref[:, 1:2]
    iot = lax.broadcasted_iota(jnp.int32, (EDGE_B, NUM_REL2), 1).astype(F32)
    onehot = (et == iot).astype(F32)
    relrows = jnp.dot(onehot, rel_ref[...], preferred_element_type=F32)
    ep = jnp.dot(hs * relrows, wnr_ref[...], preferred_element_type=F32)
    att16 = jnp.dot(ep, a1_ref[...], preferred_element_type=F32) + sa_ref[...]
    kmask = (lax.broadcasted_iota(jnp.int32, (EDGE_B, 16), 1) < K).astype(F32)
    att16 = jnp.maximum(att16, 0.0) * kmask
    m = jnp.max(att16, axis=1, keepdims=True)
    e = jnp.exp(att16 - m) * kmask
    att = e / jnp.sum(e, axis=1, keepdims=True)
    full = jnp.dot(att, rrep_ref[...], preferred_element_type=F32)
    scaled = ep * full * aux_ref[:, 0:1]
    out_ref[...] = jnp.dot(scaled, bd_ref[0], preferred_element_type=F32)


def _scatter_body(msg_hbm, dst_hbm, parts_hbm, zbuf_v, idx_v, rows_v, accum_sh,
                  sem):
    c = lax.axis_index("c")
    s = lax.axis_index("s")
    wid = c * NS + s
    ebase = wid * E_PER_W
    rbase = s * V_PER_T

    def zrow(r, carry):
        for jj in range(IN_C // 16):
            zbuf_v[r, pl.ds(jj * 16, 16)] = jnp.zeros((16,), F32)
        return carry

    lax.fori_loop(0, ZROWS, zrow, 0)
    for t in range(V_PER_T // ZROWS):
        pltpu.sync_copy(zbuf_v, accum_sh.at[pl.ds(rbase + t * ZROWS, ZROWS)])
    plsc.subcore_barrier()

    def step(j, carry):
        base = ebase + j * CHUNK
        pltpu.sync_copy(dst_hbm.at[pl.ds(base, CHUNK)], idx_v)
        pltpu.sync_copy(msg_hbm.at[pl.ds(base, CHUNK)], rows_v)
        pltpu.sync_copy(rows_v, accum_sh.at[idx_v], add=True)
        return carry

    lax.fori_loop(0, N_CHUNK, step, 0)
    plsc.subcore_barrier()

    for t in range(V_PER_T // ZROWS):
        r = rbase + t * ZROWS
        pltpu.sync_copy(accum_sh.at[pl.ds(r, ZROWS)],
                        parts_hbm.at[c, pl.ds(r, ZROWS)])


def _final_body(parts_ref, lrel_ref, lw_ref, bias_ref, gamma_ref, beta_ref,
                out_ref):
    h = (parts_ref[0, :V] + parts_ref[1, :V]) * (1.0 / 3.0)
    loop = jnp.dot(h * lrel_ref[...], lw_ref[...], preferred_element_type=F32)
    h = h + loop * (1.0 / 3.0) + bias_ref[...]
    mean = jnp.mean(h, axis=0, keepdims=True)
    var = jnp.mean((h - mean) * (h - mean), axis=0, keepdims=True)
    h = (h - mean) * jax.lax.rsqrt(var + 1e-5) * gamma_ref[...] + beta_ref[...]
    out_ref[...] = jnp.maximum(h, 0.0)


def kernel(node_repr, rel_repr, edge_norm, hyper_plane_node_w,
           hyper_plane_node_rel_w, hyper_plane_in_w, hyper_plane_out_w,
           loop_w, loop_rel, att_w, w_rel, bias, bn_gamma, bn_beta,
           edge_index, edge_type):
    assert node_repr.shape == (V, IN_C) and edge_index.shape == (2, E)

    wnr_flat = jnp.transpose(hyper_plane_node_rel_w, (1, 0, 2)).reshape(IN_C, K * D)
    wn_flat = jnp.transpose(hyper_plane_node_w, (1, 0, 2)).reshape(IN_C, K * D)
    a1 = att_w[:D, 0]
    a2 = att_w[D:, 0]
    kk = jnp.arange(K * D) // D
    jj = jnp.arange(K * D) % D
    cols16 = jnp.arange(16)
    a1_blk = jnp.where(kk[:, None] == cols16[None, :], a1[jj][:, None], 0.0)
    a2_blk = jnp.where(kk[:, None] == cols16[None, :], a2[jj][:, None], 0.0)
    rrep = jnp.where(cols16[:, None] == kk[None, :], 1.0, 0.0).astype(F32)
    oo = jnp.arange(OUT_C) // (OUT_C // K)
    oo_mod = jnp.arange(OUT_C) % (OUT_C // K)
    blkmask = (kk[:, None] == oo[None, :]).astype(F32)
    win_full = hyper_plane_in_w[kk, jj][:, oo_mod]
    wout_full = hyper_plane_out_w[kk, jj][:, oo_mod]
    bd = jnp.stack([win_full * blkmask, wout_full * blkmask])

    src = edge_index[0].astype(jnp.int32)
    dst = edge_index[1].astype(jnp.int32)
    aux = jnp.concatenate(
        [edge_norm[:, None], edge_type.astype(F32)[:, None]], axis=1)

    s16, rel_out = pl.pallas_call(
        _prep_body,
        out_shape=[jax.ShapeDtypeStruct((V, 16), F32),
                   jax.ShapeDtypeStruct((NUM_REL2, OUT_C), F32)],
    )(node_repr, wn_flat, a2_blk, rel_repr, w_rel)

    mesh = plsc.VectorSubcoreMesh(core_axis_name="c", subcore_axis_name="s")
    gather = pl.kernel(
        _gather_body,
        out_type=[jax.ShapeDtypeStruct((E, IN_C), F32),
                  jax.ShapeDtypeStruct((E, 16), F32)],
        mesh=mesh,
        scratch_types=[
            pltpu.VMEM((CHUNK,), jnp.int32),
            pltpu.VMEM((CHUNK,), jnp.int32),
            pltpu.VMEM((CHUNK, IN_C), F32),
            pltpu.VMEM((CHUNK, 16), F32),
            pltpu.SemaphoreType.DMA,
        ],
        compiler_params=pltpu.CompilerParams(use_tc_tiling_on_sc=False),
    )
    hsrc, sa = gather(node_repr, s16, src, dst)

    msg = pl.pallas_call(
        _edge_body,
        grid=(N_EB,),
        in_specs=[
            pl.BlockSpec((EDGE_B, IN_C), lambda i: (i, 0)),
            pl.BlockSpec((EDGE_B, 16), lambda i: (i, 0)),
            pl.BlockSpec((EDGE_B, 2), lambda i: (i, 0)),
            pl.BlockSpec((NUM_REL2, IN_C), lambda i: (0, 0)),
            pl.BlockSpec((IN_C, K * D), lambda i: (0, 0)),
            pl.BlockSpec((IN_C, 16), lambda i: (0, 0)),
            pl.BlockSpec((16, IN_C), lambda i: (0, 0)),
            pl.BlockSpec((1, IN_C, OUT_C), lambda i: (i // (N_EB // 2), 0, 0)),
        ],
        out_specs=pl.BlockSpec((EDGE_B, OUT_C), lambda i: (i, 0)),
        out_shape=jax.ShapeDtypeStruct((E, OUT_C), F32),
        compiler_params=pltpu.CompilerParams(
            dimension_semantics=("arbitrary",)),
    )(hsrc, sa, aux, rel_repr, wnr_flat, a1_blk, rrep, bd)

    scatter = pl.kernel(
        _scatter_body,
        out_type=jax.ShapeDtypeStruct((NC, V_PAD, OUT_C), F32),
        mesh=mesh,
        scratch_types=[
            pltpu.VMEM((ZROWS, OUT_C), F32),
            pltpu.VMEM((CHUNK,), jnp.int32),
            pltpu.VMEM((CHUNK, OUT_C), F32),
            pltpu.VMEM_SHARED((V_PAD, OUT_C), F32),
            pltpu.SemaphoreType.DMA,
        ],
    )
    parts = scatter(msg, dst)

    h = pl.pallas_call(
        _final_body,
        out_shape=jax.ShapeDtypeStruct((V, OUT_C), F32),
    )(parts, loop_rel, loop_w, bias[None, :], bn_gamma[None, :],
      bn_beta[None, :])

    return h, rel_out
